# baseline (speedup 1.0000x reference)
import functools

import jax
import jax.numpy as jnp
from jax import lax
from jax.experimental import pallas as pl
from jax.experimental.pallas import tpu as pltpu
from jax.experimental.pallas import tpu_sc as plsc

N = 10000
H = 128
E = 320000
NW = 32
LANES = 16
VROWS = 80
EPAD = NW * VROWS * 128
NP = 10240


def _tc_scores_body(w_ref, h_ref, b_ref, a_ref, c_ref, s_ref):
    for k in range(8):
        hk = h_ref[pl.ds(k * 128, 128), :]
        acc = jax.lax.dot_general(
            w_ref[...], hk, (((1,), (1,)), ((), ())),
            preferred_element_type=jnp.float32) + b_ref[:, :1]
        a_ref[k:k + 1, :] = acc[0:1, :]
        c_ref[k:k + 1, :] = acc[1:2, :]
        s_ref[k:k + 1, :] = acc[2:3, :]


def _tc_softmax_body(p0_ref, p1_ref, sb_ref, o_ref):
    x = p0_ref[...] + p1_ref[...] + sb_ref[...]
    ridx = lax.broadcasted_iota(jnp.int32, x.shape, 0)
    lidx = lax.broadcasted_iota(jnp.int32, x.shape, 1)
    valid = ridx * 128 + lidx < N
    x = jnp.where(valid, x, -jnp.inf)
    m = jnp.max(x)
    e = jnp.exp(x - m)
    s = jnp.sum(e)
    o_ref[...] = e * (1.0 / s)


def _sc_edge_body(a_hbm, c_hbm, srcm_hbm, dstm_hbm, z_hbm, out_hbm,
                  a_v, c_v, src_v, dst_v, vals_v, acc_sh, sem):
    cid = lax.axis_index("c")
    sid = lax.axis_index("s")
    wid = sid * 2 + cid

    pltpu.sync_copy(a_hbm, a_v)
    pltpu.sync_copy(c_hbm, c_v)
    pltpu.sync_copy(srcm_hbm.at[wid], src_v)
    pltpu.sync_copy(dstm_hbm.at[wid], dst_v)

    @pl.when(sid == 0)
    def _init():
        pltpu.sync_copy(z_hbm, acc_sh)

    def row(r, carry):
        for l in range(128 // LANES):
            si = src_v[r, pl.ds(l * LANES, LANES)]
            di = dst_v[r, pl.ds(l * LANES, LANES)]
            va = plsc.load_gather(a_v, [si])
            vc = plsc.load_gather(c_v, [di])
            vals_v[r, pl.ds(l * LANES, LANES)] = va + vc
        return carry

    lax.fori_loop(0, VROWS, row, 0)

    plsc.subcore_barrier()

    for r0 in range(0, VROWS, 8):
        hi = min(r0 + 8, VROWS)
        cps = [pltpu.async_copy(vals_v.at[r], acc_sh.at[dst_v.at[r]], sem,
                                add=True)
               for r in range(r0, hi)]
        for cp in cps:
            cp.wait()

    plsc.subcore_barrier()

    @pl.when(sid == 0)
    def _flush():
        pltpu.sync_copy(acc_sh, out_hbm.at[cid])


@functools.cache
def _sc_edge():
    return pl.kernel(
        _sc_edge_body,
        out_type=jax.ShapeDtypeStruct((2, NP), jnp.float32),
        mesh=plsc.VectorSubcoreMesh(core_axis_name="c", subcore_axis_name="s"),
        compiler_params=pltpu.CompilerParams(needs_layout_passes=False),
        scratch_types=[
            pltpu.VMEM((NP,), jnp.float32),
            pltpu.VMEM((NP,), jnp.float32),
            pltpu.VMEM((VROWS, 128), jnp.int32),
            pltpu.VMEM((VROWS, 128), jnp.int32),
            pltpu.VMEM((VROWS, 128), jnp.float32),
            pltpu.MemorySpace.VMEM_SHARED((NP,), jnp.float32),
            pltpu.SemaphoreType.DMA,
        ],
    )


@jax.jit
def kernel(h, edge_index, W_edge, b_edge, W_node, b_node):
    h = h.astype(jnp.float32)
    ei = edge_index.astype(jnp.int32)
    pad = jnp.full((EPAD - E,), N, dtype=jnp.int32)
    srcm = jnp.concatenate([ei[0], pad]).reshape(NW, VROWS, 128)
    dstm = jnp.concatenate([ei[1], pad]).reshape(NW, VROWS, 128)

    w3 = jnp.zeros((8, H), jnp.float32)
    w3 = w3.at[0].set(W_edge[:H]).at[1].set(W_edge[H:]).at[2].set(W_node)
    bias = jnp.zeros((8, 1), jnp.float32)
    bias = bias.at[1, 0].set(b_edge).at[2, 0].set(b_node)
    bias = jnp.broadcast_to(bias, (8, 128))

    blk = jax.ShapeDtypeStruct((NP // 128, 128), jnp.float32)
    a80, c80, s80 = pl.pallas_call(
        _tc_scores_body,
        grid=(NP // 1024,),
        in_specs=[
            pl.BlockSpec((8, 128), lambda j: (0, 0)),
            pl.BlockSpec((1024, 128), lambda j: (j, 0)),
            pl.BlockSpec((8, 128), lambda j: (0, 0)),
        ],
        out_specs=[
            pl.BlockSpec((8, 128), lambda j: (j, 0)),
            pl.BlockSpec((8, 128), lambda j: (j, 0)),
            pl.BlockSpec((8, 128), lambda j: (j, 0)),
        ],
        out_shape=[blk, blk, blk],
    )(w3, h, bias)

    zeros = jnp.zeros((NP,), jnp.float32)
    parts = _sc_edge()(a80.reshape(NP), c80.reshape(NP), srcm, dstm, zeros)

    p0 = parts[0].reshape(NP // 128, 128)
    p1 = parts[1].reshape(NP // 128, 128)

    out = pl.pallas_call(
        _tc_softmax_body,
        out_shape=jax.ShapeDtypeStruct((NP // 128, 128), jnp.float32),
    )(p0, p1, s80)

    return out.reshape(NP)[:N]


# speedup vs baseline: 1.0066x; 1.0066x over previous
import functools

import jax
import jax.numpy as jnp
from jax import lax
from jax.experimental import pallas as pl
from jax.experimental.pallas import tpu as pltpu
from jax.experimental.pallas import tpu_sc as plsc

N = 10000
H = 128
E = 320000
NW = 32
LANES = 16
VROWS = 80
EPAD = NW * VROWS * 128
NP = 10240


def _tc_scores_body(w_ref, h_ref, b_ref, a_ref, c_ref, s_ref):
    for k in range(8):
        hk = h_ref[pl.ds(k * 128, 128), :]
        acc = jax.lax.dot_general(
            w_ref[...], hk, (((1,), (1,)), ((), ())),
            preferred_element_type=jnp.float32) + b_ref[:, :1]
        a_ref[k:k + 1, :] = acc[0:1, :]
        c_ref[k:k + 1, :] = acc[1:2, :]
        s_ref[k:k + 1, :] = acc[2:3, :]


def _tc_softmax_body(p0_ref, p1_ref, sb_ref, o_ref):
    x = p0_ref[...] + p1_ref[...] + sb_ref[...]
    ridx = lax.broadcasted_iota(jnp.int32, x.shape, 0)
    lidx = lax.broadcasted_iota(jnp.int32, x.shape, 1)
    valid = ridx * 128 + lidx < N
    x = jnp.where(valid, x, -jnp.inf)
    m = jnp.max(x)
    e = jnp.exp(x - m)
    s = jnp.sum(e)
    o_ref[...] = e * (1.0 / s)


def _sc_edge_body(a_hbm, c_hbm, srcf_hbm, dstf_hbm, z_hbm, out_hbm,
                  a_v, c_v, src_v, dst_v, vals_v, acc_sh, sem):
    cid = lax.axis_index("c")
    sid = lax.axis_index("s")
    wid = sid * 2 + cid

    epw = VROWS * 128
    base = wid * epw
    pltpu.sync_copy(a_hbm, a_v)
    pltpu.sync_copy(c_hbm, c_v)
    pltpu.sync_copy(srcf_hbm.at[pl.ds(base, epw)], src_v)
    pltpu.sync_copy(dstf_hbm.at[pl.ds(base, epw)], dst_v)

    @pl.when(sid == 0)
    def _init():
        pltpu.sync_copy(z_hbm, acc_sh)

    def grp(i, carry):
        for l in range(4):
            o = i * 4 * LANES + l * LANES
            si = src_v[pl.ds(o, LANES)]
            di = dst_v[pl.ds(o, LANES)]
            va = plsc.load_gather(a_v, [si])
            vc = plsc.load_gather(c_v, [di])
            vals_v[pl.ds(o, LANES)] = va + vc
        return carry

    lax.fori_loop(0, epw // (4 * LANES), grp, 0)

    plsc.subcore_barrier()

    pltpu.sync_copy(vals_v, acc_sh.at[dst_v], add=True)

    plsc.subcore_barrier()

    @pl.when(sid == 0)
    def _flush():
        pltpu.sync_copy(acc_sh, out_hbm.at[cid])


@functools.cache
def _sc_edge():
    return pl.kernel(
        _sc_edge_body,
        out_type=jax.ShapeDtypeStruct((2, NP), jnp.float32),
        mesh=plsc.VectorSubcoreMesh(core_axis_name="c", subcore_axis_name="s"),
        compiler_params=pltpu.CompilerParams(needs_layout_passes=False),
        scratch_types=[
            pltpu.VMEM((NP,), jnp.float32),
            pltpu.VMEM((NP,), jnp.float32),
            pltpu.VMEM((VROWS * 128,), jnp.int32),
            pltpu.VMEM((VROWS * 128,), jnp.int32),
            pltpu.VMEM((VROWS * 128,), jnp.float32),
            pltpu.MemorySpace.VMEM_SHARED((NP,), jnp.float32),
            pltpu.SemaphoreType.DMA,
        ],
    )


@jax.jit
def kernel(h, edge_index, W_edge, b_edge, W_node, b_node):
    h = h.astype(jnp.float32)
    ei = edge_index.astype(jnp.int32)
    pad = jnp.full((EPAD - E,), N, dtype=jnp.int32)
    srcm = jnp.concatenate([ei[0], pad])
    dstm = jnp.concatenate([ei[1], pad])

    w3 = jnp.zeros((8, H), jnp.float32)
    w3 = w3.at[0].set(W_edge[:H]).at[1].set(W_edge[H:]).at[2].set(W_node)
    bias = jnp.zeros((8, 1), jnp.float32)
    bias = bias.at[1, 0].set(b_edge).at[2, 0].set(b_node)
    bias = jnp.broadcast_to(bias, (8, 128))

    blk = jax.ShapeDtypeStruct((NP // 128, 128), jnp.float32)
    a80, c80, s80 = pl.pallas_call(
        _tc_scores_body,
        grid=(NP // 1024,),
        in_specs=[
            pl.BlockSpec((8, 128), lambda j: (0, 0)),
            pl.BlockSpec((1024, 128), lambda j: (j, 0)),
            pl.BlockSpec((8, 128), lambda j: (0, 0)),
        ],
        out_specs=[
            pl.BlockSpec((8, 128), lambda j: (j, 0)),
            pl.BlockSpec((8, 128), lambda j: (j, 0)),
            pl.BlockSpec((8, 128), lambda j: (j, 0)),
        ],
        out_shape=[blk, blk, blk],
    )(w3, h, bias)

    zeros = jnp.zeros((NP,), jnp.float32)
    parts = _sc_edge()(a80.reshape(NP), c80.reshape(NP), srcm, dstm, zeros)

    p0 = parts[0].reshape(NP // 128, 128)
    p1 = parts[1].reshape(NP // 128, 128)

    out = pl.pallas_call(
        _tc_softmax_body,
        out_shape=jax.ShapeDtypeStruct((NP // 128, 128), jnp.float32),
    )(p0, p1, s80)

    return out.reshape(NP)[:N]


# speedup vs baseline: 1.3562x; 1.3473x over previous
import functools

import jax
import jax.numpy as jnp
from jax import lax
from jax.experimental import pallas as pl
from jax.experimental.pallas import tpu as pltpu
from jax.experimental.pallas import tpu_sc as plsc

N = 10000
H = 128
E = 320000
NW = 32
LANES = 16
TCOLS = 2500
TPW = 79
EPW = TPW * 128
NP = 10240
SENT = N + 16


def _tc_scores_body(w_ref, h_ref, b_ref, a_ref, c_ref, s_ref):
    for k in range(8):
        hk = h_ref[pl.ds(k * 128, 128), :]
        acc = jax.lax.dot_general(
            w_ref[...], hk, (((1,), (1,)), ((), ())),
            preferred_element_type=jnp.float32) + b_ref[:, :1]
        a_ref[k:k + 1, :] = acc[0:1, :]
        c_ref[k:k + 1, :] = acc[1:2, :]
        s_ref[k:k + 1, :] = acc[2:3, :]


def _tc_softmax_body(p0_ref, p1_ref, sb_ref, o_ref):
    x = p0_ref[...] + p1_ref[...] + sb_ref[...]
    ridx = lax.broadcasted_iota(jnp.int32, x.shape, 0)
    lidx = lax.broadcasted_iota(jnp.int32, x.shape, 1)
    valid = ridx * 128 + lidx < N
    x = jnp.where(valid, x, -jnp.inf)
    m = jnp.max(x)
    e = jnp.exp(x - m)
    s = jnp.sum(e)
    o_ref[...] = e * (1.0 / s)


def _sc_edge_body(a_hbm, c_hbm, ei_hbm, z_hbm, out_hbm,
                  a_v, c_v, ei_v, dst_c, vals_v, acc_sh, sem):
    cid = lax.axis_index("c")
    sid = lax.axis_index("s")
    wid = sid * 2 + cid

    t0 = pl.multiple_of((78 * wid + jnp.minimum(wid, 4)) * 128, 128)

    pltpu.sync_copy(a_hbm, a_v)
    pltpu.sync_copy(c_hbm, c_v)

    @pl.when(wid < 4)
    def _stage_full():
        pltpu.sync_copy(ei_hbm.at[:, pl.ds(t0, EPW)], ei_v)

    @pl.when(wid >= 4)
    def _stage_part():
        pltpu.sync_copy(ei_hbm.at[:, pl.ds(t0, EPW - 128)],
                        ei_v.at[:, pl.ds(0, EPW - 128)])
        sent = jnp.full((LANES,), SENT, jnp.int32)
        for l in range(128 // LANES):
            ei_v[0, pl.ds(EPW - 128 + l * LANES, LANES)] = sent
            ei_v[1, pl.ds(EPW - 128 + l * LANES, LANES)] = sent

    @pl.when(sid == 0)
    def _init():
        pltpu.sync_copy(z_hbm, acc_sh)

    def grp(i, carry):
        for l in range(4):
            o = i * 4 * LANES + l * LANES
            si = ei_v[0, pl.ds(o, LANES)]
            di = ei_v[1, pl.ds(o, LANES)]
            va = plsc.load_gather(a_v, [si])
            vc = plsc.load_gather(c_v, [di])
            vals_v[pl.ds(o, LANES)] = va + vc
            dst_c[pl.ds(o, LANES)] = di
        return carry

    lax.fori_loop(0, EPW // (4 * LANES), grp, 0)

    plsc.subcore_barrier()

    pltpu.sync_copy(vals_v, acc_sh.at[dst_c], add=True)

    plsc.subcore_barrier()

    @pl.when(sid == 0)
    def _flush():
        pltpu.sync_copy(acc_sh, out_hbm.at[cid])


@functools.cache
def _sc_edge():
    return pl.kernel(
        _sc_edge_body,
        out_type=jax.ShapeDtypeStruct((2, NP), jnp.float32),
        mesh=plsc.VectorSubcoreMesh(core_axis_name="c", subcore_axis_name="s"),
        compiler_params=pltpu.CompilerParams(needs_layout_passes=False),
        scratch_types=[
            pltpu.VMEM((NP,), jnp.float32),
            pltpu.VMEM((NP,), jnp.float32),
            pltpu.VMEM((2, EPW), jnp.int32),
            pltpu.VMEM((EPW,), jnp.int32),
            pltpu.VMEM((EPW,), jnp.float32),
            pltpu.MemorySpace.VMEM_SHARED((NP,), jnp.float32),
            pltpu.SemaphoreType.DMA,
        ],
    )


@jax.jit
def kernel(h, edge_index, W_edge, b_edge, W_node, b_node):
    h = h.astype(jnp.float32)
    ei = edge_index.astype(jnp.int32)

    w3 = jnp.zeros((8, H), jnp.float32)
    w3 = w3.at[0].set(W_edge[:H]).at[1].set(W_edge[H:]).at[2].set(W_node)
    bias = jnp.zeros((8, 1), jnp.float32)
    bias = bias.at[1, 0].set(b_edge).at[2, 0].set(b_node)
    bias = jnp.broadcast_to(bias, (8, 128))

    blk = jax.ShapeDtypeStruct((NP // 128, 128), jnp.float32)
    a80, c80, s80 = pl.pallas_call(
        _tc_scores_body,
        grid=(NP // 1024,),
        in_specs=[
            pl.BlockSpec((8, 128), lambda j: (0, 0)),
            pl.BlockSpec((1024, 128), lambda j: (j, 0)),
            pl.BlockSpec((8, 128), lambda j: (0, 0)),
        ],
        out_specs=[
            pl.BlockSpec((8, 128), lambda j: (j, 0)),
            pl.BlockSpec((8, 128), lambda j: (j, 0)),
            pl.BlockSpec((8, 128), lambda j: (j, 0)),
        ],
        out_shape=[blk, blk, blk],
    )(w3, h, bias)

    zeros = jnp.zeros((NP,), jnp.float32)
    parts = _sc_edge()(a80.reshape(NP), c80.reshape(NP), ei, zeros)

    p0 = parts[0].reshape(NP // 128, 128)
    p1 = parts[1].reshape(NP // 128, 128)

    out = pl.pallas_call(
        _tc_softmax_body,
        out_shape=jax.ShapeDtypeStruct((NP // 128, 128), jnp.float32),
    )(p0, p1, s80)

    return out.reshape(NP)[:N]


# speedup vs baseline: 1.3698x; 1.0100x over previous
import functools

import jax
import jax.numpy as jnp
from jax import lax
from jax.experimental import pallas as pl
from jax.experimental.pallas import tpu as pltpu
from jax.experimental.pallas import tpu_sc as plsc

N = 10000
H = 128
E = 320000
NW = 32
LANES = 16
TCOLS = 2500
TPW = 79
EPW = TPW * 128
NP = 10240
SENT = N + 16


def _tc_scores_body(w_ref, h_ref, b_ref, a_ref, c_ref, s_ref):
    for k in range(8):
        hk = h_ref[pl.ds(k * 128, 128), :]
        acc = jax.lax.dot_general(
            w_ref[...], hk, (((1,), (1,)), ((), ())),
            preferred_element_type=jnp.float32) + b_ref[:, :1]
        a_ref[k:k + 1, :] = acc[0:1, :]
        c_ref[k:k + 1, :] = acc[1:2, :]
        s_ref[k:k + 1, :] = acc[2:3, :]


def _tc_softmax_body(p0_ref, p1_ref, sb_ref, o_ref):
    x = p0_ref[...] + p1_ref[...] + sb_ref[...]
    ridx = lax.broadcasted_iota(jnp.int32, x.shape, 0)
    lidx = lax.broadcasted_iota(jnp.int32, x.shape, 1)
    valid = ridx * 128 + lidx < N
    x = jnp.where(valid, x, -jnp.inf)
    m = jnp.max(x)
    e = jnp.exp(x - m)
    s = jnp.sum(e)
    o_ref[...] = e * (1.0 / s)


def _sc_edge_body(a_hbm, c_hbm, ei_hbm, z_hbm, out_hbm,
                  a_v, c_v, ei_v, acc_l, red_v, res_v, acc_sh, sem):
    cid = lax.axis_index("c")
    sid = lax.axis_index("s")
    wid = sid * 2 + cid

    t0 = pl.multiple_of((78 * wid + jnp.minimum(wid, 4)) * 128, 128)

    pltpu.sync_copy(a_hbm, a_v)
    pltpu.sync_copy(c_hbm, c_v)

    @pl.when(wid < 4)
    def _stage_full():
        pltpu.sync_copy(ei_hbm.at[:, pl.ds(t0, EPW)], ei_v)

    @pl.when(wid >= 4)
    def _stage_part():
        pltpu.sync_copy(ei_hbm.at[:, pl.ds(t0, EPW - 128)],
                        ei_v.at[:, pl.ds(0, EPW - 128)])
        sent = jnp.full((LANES,), SENT, jnp.int32)
        for l in range(128 // LANES):
            ei_v[0, pl.ds(EPW - 128 + l * LANES, LANES)] = sent
            ei_v[1, pl.ds(EPW - 128 + l * LANES, LANES)] = sent

    pltpu.sync_copy(z_hbm, acc_l)

    def grp(i, carry):
        for l in range(4):
            o = i * 4 * LANES + l * LANES
            si = ei_v[0, pl.ds(o, LANES)]
            di = ei_v[1, pl.ds(o, LANES)]
            va = plsc.load_gather(a_v, [si])
            vc = plsc.load_gather(c_v, [di])
            plsc.addupdate_scatter(acc_l, [di], va + vc)
        return carry

    lax.fori_loop(0, EPW // (4 * LANES), grp, 0)

    pltpu.sync_copy(acc_l, acc_sh.at[sid])
    plsc.subcore_barrier()

    seg = NP // 16
    pltpu.sync_copy(acc_sh.at[:, pl.ds(sid * seg, seg)], red_v)
    for g in range(seg // LANES):
        tot = red_v[0, pl.ds(g * LANES, LANES)]
        for r in range(1, 16):
            tot = tot + red_v[r, pl.ds(g * LANES, LANES)]
        res_v[pl.ds(g * LANES, LANES)] = tot

    pltpu.sync_copy(res_v, out_hbm.at[cid, pl.ds(sid * seg, seg)])


@functools.cache
def _sc_edge():
    return pl.kernel(
        _sc_edge_body,
        out_type=jax.ShapeDtypeStruct((2, NP), jnp.float32),
        mesh=plsc.VectorSubcoreMesh(core_axis_name="c", subcore_axis_name="s"),
        compiler_params=pltpu.CompilerParams(needs_layout_passes=False),
        scratch_types=[
            pltpu.VMEM((NP,), jnp.float32),
            pltpu.VMEM((NP,), jnp.float32),
            pltpu.VMEM((2, EPW), jnp.int32),
            pltpu.VMEM((NP,), jnp.float32),
            pltpu.VMEM((16, NP // 16), jnp.float32),
            pltpu.VMEM((NP // 16,), jnp.float32),
            pltpu.MemorySpace.VMEM_SHARED((16, NP), jnp.float32),
            pltpu.SemaphoreType.DMA,
        ],
    )


@jax.jit
def kernel(h, edge_index, W_edge, b_edge, W_node, b_node):
    h = h.astype(jnp.float32)
    ei = edge_index.astype(jnp.int32)

    w3 = jnp.zeros((8, H), jnp.float32)
    w3 = w3.at[0].set(W_edge[:H]).at[1].set(W_edge[H:]).at[2].set(W_node)
    bias = jnp.zeros((8, 1), jnp.float32)
    bias = bias.at[1, 0].set(b_edge).at[2, 0].set(b_node)
    bias = jnp.broadcast_to(bias, (8, 128))

    blk = jax.ShapeDtypeStruct((NP // 128, 128), jnp.float32)
    a80, c80, s80 = pl.pallas_call(
        _tc_scores_body,
        grid=(NP // 1024,),
        in_specs=[
            pl.BlockSpec((8, 128), lambda j: (0, 0)),
            pl.BlockSpec((1024, 128), lambda j: (j, 0)),
            pl.BlockSpec((8, 128), lambda j: (0, 0)),
        ],
        out_specs=[
            pl.BlockSpec((8, 128), lambda j: (j, 0)),
            pl.BlockSpec((8, 128), lambda j: (j, 0)),
            pl.BlockSpec((8, 128), lambda j: (j, 0)),
        ],
        out_shape=[blk, blk, blk],
    )(w3, h, bias)

    zeros = jnp.zeros((NP,), jnp.float32)
    parts = _sc_edge()(a80.reshape(NP), c80.reshape(NP), ei, zeros)

    p0 = parts[0].reshape(NP // 128, 128)
    p1 = parts[1].reshape(NP // 128, 128)

    out = pl.pallas_call(
        _tc_softmax_body,
        out_shape=jax.ShapeDtypeStruct((NP // 128, 128), jnp.float32),
    )(p0, p1, s80)

    return out.reshape(NP)[:N]


# speedup vs baseline: 1.4162x; 1.0338x over previous
import functools

import jax
import jax.numpy as jnp
from jax import lax
from jax.experimental import pallas as pl
from jax.experimental.pallas import tpu as pltpu
from jax.experimental.pallas import tpu_sc as plsc

N = 10000
H = 128
E = 320000
NW = 32
LANES = 16
TCOLS = 2500
TPW = 79
EPW = TPW * 128
NP = 10240
SENT = N + 16


def _tc_scores_body(w_ref, h_ref, b_ref, a_ref, c_ref, s_ref):
    for k in range(8):
        hk = h_ref[pl.ds(k * 128, 128), :]
        acc = jax.lax.dot_general(
            w_ref[...], hk, (((1,), (1,)), ((), ())),
            preferred_element_type=jnp.float32) + b_ref[:, :1]
        a_ref[k:k + 1, :] = acc[0:1, :]
        c_ref[k:k + 1, :] = acc[1:2, :]
        s_ref[k:k + 1, :] = acc[2:3, :]


def _tc_softmax_body(p0_ref, p1_ref, sb_ref, o_ref):
    x = p0_ref[...] + p1_ref[...] + sb_ref[...]
    ridx = lax.broadcasted_iota(jnp.int32, x.shape, 0)
    lidx = lax.broadcasted_iota(jnp.int32, x.shape, 1)
    valid = ridx * 128 + lidx < N
    x = jnp.where(valid, x, -jnp.inf)
    m = jnp.max(x)
    e = jnp.exp(x - m)
    s = jnp.sum(e)
    o_ref[...] = e * (1.0 / s)


def _sc_edge_body(a_hbm, c_hbm, ei_hbm, z_hbm, out_hbm,
                  a_v, c_v, ei_v, acc_l, red_v, res_v, acc_sh, sem):
    cid = lax.axis_index("c")
    sid = lax.axis_index("s")
    wid = sid * 2 + cid

    t0 = pl.multiple_of((78 * wid + jnp.minimum(wid, 4)) * 128, 128)

    cpa = pltpu.async_copy(a_hbm, a_v, sem)
    cpc = pltpu.async_copy(c_hbm, c_v, sem)
    cpz = pltpu.async_copy(z_hbm, acc_l, sem)

    @pl.when(wid < 4)
    def _stage_full():
        pltpu.sync_copy(ei_hbm.at[:, pl.ds(t0, EPW)], ei_v)

    @pl.when(wid >= 4)
    def _stage_part():
        pltpu.sync_copy(ei_hbm.at[:, pl.ds(t0, EPW - 128)],
                        ei_v.at[:, pl.ds(0, EPW - 128)])
        sent = jnp.full((LANES,), SENT, jnp.int32)
        for l in range(128 // LANES):
            ei_v[0, pl.ds(EPW - 128 + l * LANES, LANES)] = sent
            ei_v[1, pl.ds(EPW - 128 + l * LANES, LANES)] = sent

    cpa.wait()
    cpc.wait()
    cpz.wait()

    def grp(i, carry):
        for l in range(4):
            o = i * 4 * LANES + l * LANES
            si = ei_v[0, pl.ds(o, LANES)]
            di = ei_v[1, pl.ds(o, LANES)]
            va = plsc.load_gather(a_v, [si])
            vc = plsc.load_gather(c_v, [di])
            plsc.addupdate_scatter(acc_l, [di], va + vc)
        return carry

    lax.fori_loop(0, EPW // (4 * LANES), grp, 0)

    pltpu.sync_copy(acc_l, acc_sh.at[sid])
    plsc.subcore_barrier()

    seg = NP // 16
    pltpu.sync_copy(acc_sh.at[:, pl.ds(sid * seg, seg)], red_v)
    for g in range(seg // LANES):
        tot = red_v[0, pl.ds(g * LANES, LANES)]
        for r in range(1, 16):
            tot = tot + red_v[r, pl.ds(g * LANES, LANES)]
        res_v[pl.ds(g * LANES, LANES)] = tot

    pltpu.sync_copy(res_v, out_hbm.at[cid, pl.ds(sid * seg, seg)])


@functools.cache
def _sc_edge():
    return pl.kernel(
        _sc_edge_body,
        out_type=jax.ShapeDtypeStruct((2, NP), jnp.float32),
        mesh=plsc.VectorSubcoreMesh(core_axis_name="c", subcore_axis_name="s"),
        compiler_params=pltpu.CompilerParams(needs_layout_passes=False),
        scratch_types=[
            pltpu.VMEM((NP,), jnp.float32),
            pltpu.VMEM((NP,), jnp.float32),
            pltpu.VMEM((2, EPW), jnp.int32),
            pltpu.VMEM((NP,), jnp.float32),
            pltpu.VMEM((16, NP // 16), jnp.float32),
            pltpu.VMEM((NP // 16,), jnp.float32),
            pltpu.MemorySpace.VMEM_SHARED((16, NP), jnp.float32),
            pltpu.SemaphoreType.DMA,
        ],
    )


@jax.jit
def kernel(h, edge_index, W_edge, b_edge, W_node, b_node):
    h = h.astype(jnp.float32)
    ei = edge_index.astype(jnp.int32)

    w3 = jnp.zeros((8, H), jnp.float32)
    w3 = w3.at[0].set(W_edge[:H]).at[1].set(W_edge[H:]).at[2].set(W_node)
    bias = jnp.zeros((8, 1), jnp.float32)
    bias = bias.at[1, 0].set(b_edge).at[2, 0].set(b_node)
    bias = jnp.broadcast_to(bias, (8, 128))

    blk = jax.ShapeDtypeStruct((NP // 128, 128), jnp.float32)
    a80, c80, s80 = pl.pallas_call(
        _tc_scores_body,
        grid=(NP // 1024,),
        in_specs=[
            pl.BlockSpec((8, 128), lambda j: (0, 0)),
            pl.BlockSpec((1024, 128), lambda j: (j, 0)),
            pl.BlockSpec((8, 128), lambda j: (0, 0)),
        ],
        out_specs=[
            pl.BlockSpec((8, 128), lambda j: (j, 0)),
            pl.BlockSpec((8, 128), lambda j: (j, 0)),
            pl.BlockSpec((8, 128), lambda j: (j, 0)),
        ],
        out_shape=[blk, blk, blk],
    )(w3, h, bias)

    zeros = jnp.zeros((NP,), jnp.float32)
    parts = _sc_edge()(a80.reshape(NP), c80.reshape(NP), ei, zeros)

    p0 = parts[0].reshape(NP // 128, 128)
    p1 = parts[1].reshape(NP // 128, 128)

    out = pl.pallas_call(
        _tc_softmax_body,
        out_shape=jax.ShapeDtypeStruct((NP // 128, 128), jnp.float32),
    )(p0, p1, s80)

    return out.reshape(NP)[:N]


# speedup vs baseline: 1.5826x; 1.1175x over previous
import functools

import jax
import jax.numpy as jnp
from jax import lax
from jax.experimental import pallas as pl
from jax.experimental.pallas import tpu as pltpu
from jax.experimental.pallas import tpu_sc as plsc

N = 10000
H = 128
E = 320000
NW = 32
LANES = 16
TCOLS = 2500
TPW = 79
EPW = TPW * 128
NP = 10240
SENT = N + 16


def _tc_scores_body(w_ref, h_ref, b_ref, a_ref, c_ref, s_ref):
    for k in range(8):
        hk = h_ref[pl.ds(k * 128, 128), :]
        acc = jax.lax.dot_general(
            w_ref[...], hk, (((1,), (1,)), ((), ())),
            preferred_element_type=jnp.float32) + b_ref[:, :1]
        a_ref[k:k + 1, :] = acc[0:1, :]
        c_ref[k:k + 1, :] = acc[1:2, :]
        s_ref[k:k + 1, :] = acc[2:3, :]


def _tc_softmax_body(p0_ref, p1_ref, sb_ref, o_ref):
    x = p0_ref[...] + p1_ref[...] + sb_ref[...]
    ridx = lax.broadcasted_iota(jnp.int32, x.shape, 0)
    lidx = lax.broadcasted_iota(jnp.int32, x.shape, 1)
    valid = ridx * 128 + lidx < N
    x = jnp.where(valid, x, -jnp.inf)
    m = jnp.max(x)
    e = jnp.exp(x - m)
    s = jnp.sum(e)
    o_ref[...] = e * (1.0 / s)


def _sc_edge_body(a_hbm, c_hbm, ei_hbm, z_hbm, out_hbm,
                  a_v, c_v, ei_v, acc_l, red_v, res_v, acc_sh, sem):
    cid = lax.axis_index("c")
    sid = lax.axis_index("s")
    wid = sid * 2 + cid

    t0 = pl.multiple_of((78 * wid + jnp.minimum(wid, 4)) * 128, 128)

    cpa = pltpu.async_copy(a_hbm, a_v, sem)
    cpc = pltpu.async_copy(c_hbm, c_v, sem)
    cpz = pltpu.async_copy(z_hbm, acc_l, sem)

    @pl.when(wid < 4)
    def _stage_full():
        pltpu.sync_copy(ei_hbm.at[:, pl.ds(t0, EPW)], ei_v)

    @pl.when(wid >= 4)
    def _stage_part():
        pltpu.sync_copy(ei_hbm.at[:, pl.ds(t0, EPW - 128)],
                        ei_v.at[:, pl.ds(0, EPW - 128)])
        sent = jnp.full((LANES,), SENT, jnp.int32)
        for l in range(128 // LANES):
            ei_v[0, pl.ds(EPW - 128 + l * LANES, LANES)] = sent
            ei_v[1, pl.ds(EPW - 128 + l * LANES, LANES)] = sent

    cpa.wait()
    cpc.wait()
    cpz.wait()

    @plsc.parallel_loop(0, EPW, LANES, unroll=8)
    def _grp(o):
        si = ei_v[0, pl.ds(o, LANES)]
        di = ei_v[1, pl.ds(o, LANES)]
        va = plsc.load_gather(a_v, [si])
        vc = plsc.load_gather(c_v, [di])
        plsc.addupdate_scatter(acc_l, [di], va + vc)

    pltpu.sync_copy(acc_l, acc_sh.at[sid])
    plsc.subcore_barrier()

    seg = NP // 16
    pltpu.sync_copy(acc_sh.at[:, pl.ds(sid * seg, seg)], red_v)
    for g in range(seg // LANES):
        tot = red_v[0, pl.ds(g * LANES, LANES)]
        for r in range(1, 16):
            tot = tot + red_v[r, pl.ds(g * LANES, LANES)]
        res_v[pl.ds(g * LANES, LANES)] = tot

    pltpu.sync_copy(res_v, out_hbm.at[cid, pl.ds(sid * seg, seg)])


@functools.cache
def _sc_edge():
    return pl.kernel(
        _sc_edge_body,
        out_type=jax.ShapeDtypeStruct((2, NP), jnp.float32),
        mesh=plsc.VectorSubcoreMesh(core_axis_name="c", subcore_axis_name="s"),
        compiler_params=pltpu.CompilerParams(needs_layout_passes=False),
        scratch_types=[
            pltpu.VMEM((NP,), jnp.float32),
            pltpu.VMEM((NP,), jnp.float32),
            pltpu.VMEM((2, EPW), jnp.int32),
            pltpu.VMEM((NP,), jnp.float32),
            pltpu.VMEM((16, NP // 16), jnp.float32),
            pltpu.VMEM((NP // 16,), jnp.float32),
            pltpu.MemorySpace.VMEM_SHARED((16, NP), jnp.float32),
            pltpu.SemaphoreType.DMA,
        ],
    )


@jax.jit
def kernel(h, edge_index, W_edge, b_edge, W_node, b_node):
    h = h.astype(jnp.float32)
    ei = edge_index.astype(jnp.int32)

    w3 = jnp.zeros((8, H), jnp.float32)
    w3 = w3.at[0].set(W_edge[:H]).at[1].set(W_edge[H:]).at[2].set(W_node)
    bias = jnp.zeros((8, 1), jnp.float32)
    bias = bias.at[1, 0].set(b_edge).at[2, 0].set(b_node)
    bias = jnp.broadcast_to(bias, (8, 128))

    blk = jax.ShapeDtypeStruct((NP // 128, 128), jnp.float32)
    a80, c80, s80 = pl.pallas_call(
        _tc_scores_body,
        grid=(NP // 1024,),
        in_specs=[
            pl.BlockSpec((8, 128), lambda j: (0, 0)),
            pl.BlockSpec((1024, 128), lambda j: (j, 0)),
            pl.BlockSpec((8, 128), lambda j: (0, 0)),
        ],
        out_specs=[
            pl.BlockSpec((8, 128), lambda j: (j, 0)),
            pl.BlockSpec((8, 128), lambda j: (j, 0)),
            pl.BlockSpec((8, 128), lambda j: (j, 0)),
        ],
        out_shape=[blk, blk, blk],
    )(w3, h, bias)

    zeros = jnp.zeros((NP,), jnp.float32)
    parts = _sc_edge()(a80.reshape(NP), c80.reshape(NP), ei, zeros)

    p0 = parts[0].reshape(NP // 128, 128)
    p1 = parts[1].reshape(NP // 128, 128)

    out = pl.pallas_call(
        _tc_softmax_body,
        out_shape=jax.ShapeDtypeStruct((NP // 128, 128), jnp.float32),
    )(p0, p1, s80)

    return out.reshape(NP)[:N]


# speedup vs baseline: 1.5928x; 1.0065x over previous
import functools

import jax
import jax.numpy as jnp
from jax import lax
from jax.experimental import pallas as pl
from jax.experimental.pallas import tpu as pltpu
from jax.experimental.pallas import tpu_sc as plsc

N = 10000
H = 128
E = 320000
NW = 32
LANES = 16
TCOLS = 2500
TPW = 79
EPW = TPW * 128
NP = 10240
SENT = N + 16


def _tc_scores_body(w_ref, h_ref, b_ref, a_ref, c_ref, s_ref):
    acc = jax.lax.dot_general(
        w_ref[...], h_ref[...], (((1,), (1,)), ((), ())),
        preferred_element_type=jnp.float32) + b_ref[:, :1]
    for k in range(8):
        blk = acc[:, 128 * k:128 * (k + 1)]
        a_ref[k:k + 1, :] = blk[0:1, :]
        c_ref[k:k + 1, :] = blk[1:2, :]
        s_ref[k:k + 1, :] = blk[2:3, :]


def _tc_softmax_body(p0_ref, p1_ref, sb_ref, o_ref):
    x = p0_ref[...] + p1_ref[...] + sb_ref[...]
    ridx = lax.broadcasted_iota(jnp.int32, x.shape, 0)
    lidx = lax.broadcasted_iota(jnp.int32, x.shape, 1)
    valid = ridx * 128 + lidx < N
    x = jnp.where(valid, x, -jnp.inf)
    m = jnp.max(x)
    e = jnp.exp(x - m)
    s = jnp.sum(e)
    o_ref[...] = e * (1.0 / s)


def _sc_edge_body(a_hbm, c_hbm, ei_hbm, z_hbm, out_hbm,
                  a_v, c_v, ei_v, acc_l, red_v, res_v, acc_sh, sem):
    cid = lax.axis_index("c")
    sid = lax.axis_index("s")
    wid = sid * 2 + cid

    t0 = pl.multiple_of((78 * wid + jnp.minimum(wid, 4)) * 128, 128)

    cpa = pltpu.async_copy(a_hbm, a_v, sem)
    cpc = pltpu.async_copy(c_hbm, c_v, sem)
    cpz = pltpu.async_copy(z_hbm, acc_l, sem)

    @pl.when(wid < 4)
    def _stage_full():
        pltpu.sync_copy(ei_hbm.at[:, pl.ds(t0, EPW)], ei_v)

    @pl.when(wid >= 4)
    def _stage_part():
        pltpu.sync_copy(ei_hbm.at[:, pl.ds(t0, EPW - 128)],
                        ei_v.at[:, pl.ds(0, EPW - 128)])
        sent = jnp.full((LANES,), SENT, jnp.int32)
        for l in range(128 // LANES):
            ei_v[0, pl.ds(EPW - 128 + l * LANES, LANES)] = sent
            ei_v[1, pl.ds(EPW - 128 + l * LANES, LANES)] = sent

    cpa.wait()
    cpc.wait()
    cpz.wait()

    @plsc.parallel_loop(0, EPW, LANES, unroll=8)
    def _grp(o):
        si = ei_v[0, pl.ds(o, LANES)]
        di = ei_v[1, pl.ds(o, LANES)]
        va = plsc.load_gather(a_v, [si])
        vc = plsc.load_gather(c_v, [di])
        plsc.addupdate_scatter(acc_l, [di], va + vc)

    pltpu.sync_copy(acc_l, acc_sh.at[sid])
    plsc.subcore_barrier()

    seg = NP // 16
    pltpu.sync_copy(acc_sh.at[:, pl.ds(sid * seg, seg)], red_v)
    for g in range(seg // LANES):
        tot = red_v[0, pl.ds(g * LANES, LANES)]
        for r in range(1, 16):
            tot = tot + red_v[r, pl.ds(g * LANES, LANES)]
        res_v[pl.ds(g * LANES, LANES)] = tot

    pltpu.sync_copy(res_v, out_hbm.at[cid, pl.ds(sid * seg, seg)])


@functools.cache
def _sc_edge():
    return pl.kernel(
        _sc_edge_body,
        out_type=jax.ShapeDtypeStruct((2, NP), jnp.float32),
        mesh=plsc.VectorSubcoreMesh(core_axis_name="c", subcore_axis_name="s"),
        compiler_params=pltpu.CompilerParams(needs_layout_passes=False),
        scratch_types=[
            pltpu.VMEM((NP,), jnp.float32),
            pltpu.VMEM((NP,), jnp.float32),
            pltpu.VMEM((2, EPW), jnp.int32),
            pltpu.VMEM((NP,), jnp.float32),
            pltpu.VMEM((16, NP // 16), jnp.float32),
            pltpu.VMEM((NP // 16,), jnp.float32),
            pltpu.MemorySpace.VMEM_SHARED((16, NP), jnp.float32),
            pltpu.SemaphoreType.DMA,
        ],
    )


@jax.jit
def kernel(h, edge_index, W_edge, b_edge, W_node, b_node):
    h = h.astype(jnp.float32)
    ei = edge_index.astype(jnp.int32)

    w3 = jnp.zeros((8, H), jnp.float32)
    w3 = w3.at[0].set(W_edge[:H]).at[1].set(W_edge[H:]).at[2].set(W_node)
    bias = jnp.zeros((8, 1), jnp.float32)
    bias = bias.at[1, 0].set(b_edge).at[2, 0].set(b_node)
    bias = jnp.broadcast_to(bias, (8, 128))

    blk = jax.ShapeDtypeStruct((NP // 128, 128), jnp.float32)
    a80, c80, s80 = pl.pallas_call(
        _tc_scores_body,
        grid=(NP // 1024,),
        in_specs=[
            pl.BlockSpec((8, 128), lambda j: (0, 0)),
            pl.BlockSpec((1024, 128), lambda j: (j, 0)),
            pl.BlockSpec((8, 128), lambda j: (0, 0)),
        ],
        out_specs=[
            pl.BlockSpec((8, 128), lambda j: (j, 0)),
            pl.BlockSpec((8, 128), lambda j: (j, 0)),
            pl.BlockSpec((8, 128), lambda j: (j, 0)),
        ],
        out_shape=[blk, blk, blk],
    )(w3, h, bias)

    zeros = jnp.zeros((NP,), jnp.float32)
    parts = _sc_edge()(a80.reshape(NP), c80.reshape(NP), ei, zeros)

    p0 = parts[0].reshape(NP // 128, 128)
    p1 = parts[1].reshape(NP // 128, 128)

    out = pl.pallas_call(
        _tc_softmax_body,
        out_shape=jax.ShapeDtypeStruct((NP // 128, 128), jnp.float32),
    )(p0, p1, s80)

    return out.reshape(NP)[:N]


# speedup vs baseline: 1.6155x; 1.0143x over previous
import functools

import jax
import jax.numpy as jnp
from jax import lax
from jax.experimental import pallas as pl
from jax.experimental.pallas import tpu as pltpu
from jax.experimental.pallas import tpu_sc as plsc

N = 10000
H = 128
E = 320000
NW = 32
LANES = 16
TPW = 79
EPW = TPW * 128
NP = 10240
SEG = 1024
SENT = N + 16


def _tc_scores_body(w_ref, h_ref, b_ref, o_ref):
    acc = jax.lax.dot_general(
        w_ref[...], h_ref[...], (((1,), (1,)), ((), ())),
        preferred_element_type=jnp.float32) + b_ref[:, :1]
    o_ref[0, :, pl.ds(0, N)] = acc[0:4, :]


def _tc_softmax_body(p_ref, sb_ref, o_ref):
    x = p_ref[0] + p_ref[1] + sb_ref[...]
    ridx = lax.broadcasted_iota(jnp.int32, x.shape, 0)
    lidx = lax.broadcasted_iota(jnp.int32, x.shape, 1)
    valid = ridx * 128 + lidx < N
    x = jnp.where(valid, x, -jnp.inf)
    m = jnp.max(x)
    e = jnp.exp(x - m)
    s = jnp.sum(e)
    o_ref[...] = e * (1.0 / s)


def _sc_edge_body(sc_hbm, ei_hbm, z_hbm, out_hbm, sb_hbm,
                  sc_v, ei_v, acc_l, red_v, res_v, sb_v, acc_sh, sem):
    cid = lax.axis_index("c")
    sid = lax.axis_index("s")
    wid = sid * 2 + cid

    t0 = pl.multiple_of((78 * wid + jnp.minimum(wid, 4)) * 128, 128)

    cps = pltpu.async_copy(sc_hbm.at[0], sc_v, sem)
    cpz = pltpu.async_copy(z_hbm, acc_l, sem)

    @pl.when(wid < 4)
    def _stage_full():
        pltpu.sync_copy(ei_hbm.at[:, pl.ds(t0, EPW)], ei_v)

    @pl.when(wid >= 4)
    def _stage_part():
        pltpu.sync_copy(ei_hbm.at[:, pl.ds(t0, EPW - 128)],
                        ei_v.at[:, pl.ds(0, EPW - 128)])
        sent = jnp.full((LANES,), SENT, jnp.int32)
        for l in range(128 // LANES):
            ei_v[0, pl.ds(EPW - 128 + l * LANES, LANES)] = sent
            ei_v[1, pl.ds(EPW - 128 + l * LANES, LANES)] = sent

    cps.wait()
    cpz.wait()

    row0 = jnp.zeros((LANES,), jnp.int32)
    row1 = jnp.full((LANES,), 1, jnp.int32)

    @plsc.parallel_loop(0, EPW, LANES, unroll=8)
    def _grp(o):
        si = ei_v[0, pl.ds(o, LANES)]
        di = ei_v[1, pl.ds(o, LANES)]
        va = plsc.load_gather(sc_v, [row0, si])
        vc = plsc.load_gather(sc_v, [row1, di])
        plsc.addupdate_scatter(acc_l, [di], va + vc)

    pltpu.sync_copy(acc_l, acc_sh.at[sid])
    plsc.subcore_barrier()

    @pl.when(sid < NP // SEG)
    def _reduce():
        pltpu.sync_copy(acc_sh.at[:, pl.ds(sid * SEG, SEG)], red_v)
        for g in range(SEG // LANES):
            tot = red_v[0, pl.ds(g * LANES, LANES)]
            for r in range(1, 16):
                tot = tot + red_v[r, pl.ds(g * LANES, LANES)]
            res_v[g // 8, pl.ds((g % 8) * LANES, LANES)] = tot

        pltpu.sync_copy(res_v, out_hbm.at[cid, pl.ds(sid * 8, 8), :])

    @pl.when((cid == 0) & (sid < NP // SEG))
    def _sb():
        for g in range(SEG // LANES):
            sb_v[g // 8, pl.ds((g % 8) * LANES, LANES)] = (
                sc_v[2, pl.ds(sid * SEG + g * LANES, LANES)])
        pltpu.sync_copy(sb_v, sb_hbm.at[pl.ds(sid * 8, 8), :])


@functools.cache
def _sc_edge():
    return pl.kernel(
        _sc_edge_body,
        out_type=(jax.ShapeDtypeStruct((2, NP // 128, 128), jnp.float32),
                  jax.ShapeDtypeStruct((NP // 128, 128), jnp.float32)),
        mesh=plsc.VectorSubcoreMesh(core_axis_name="c", subcore_axis_name="s"),
        compiler_params=pltpu.CompilerParams(needs_layout_passes=False),
        scratch_types=[
            pltpu.VMEM((4, NP), jnp.float32),
            pltpu.VMEM((2, EPW), jnp.int32),
            pltpu.VMEM((NP,), jnp.float32),
            pltpu.VMEM((16, SEG), jnp.float32),
            pltpu.VMEM((8, 128), jnp.float32),
            pltpu.VMEM((8, 128), jnp.float32),
            pltpu.MemorySpace.VMEM_SHARED((16, NP), jnp.float32),
            pltpu.SemaphoreType.DMA,
        ],
    )


@jax.jit
def kernel(h, edge_index, W_edge, b_edge, W_node, b_node):
    h = h.astype(jnp.float32)
    ei = edge_index.astype(jnp.int32)

    w3 = jnp.zeros((8, H), jnp.float32)
    w3 = w3.at[0].set(W_edge[:H]).at[1].set(W_edge[H:]).at[2].set(W_node)
    bias = jnp.zeros((8, 1), jnp.float32)
    bias = bias.at[1, 0].set(b_edge).at[2, 0].set(b_node)
    bias = jnp.broadcast_to(bias, (8, 128))

    scores = pl.pallas_call(
        _tc_scores_body,
        out_shape=jax.ShapeDtypeStruct((2, 4, NP), jnp.float32),
    )(w3, h, bias)

    zeros = jnp.zeros((NP,), jnp.float32)
    parts, sb = _sc_edge()(scores, ei, zeros)

    out = pl.pallas_call(
        _tc_softmax_body,
        out_shape=jax.ShapeDtypeStruct((NP // 128, 128), jnp.float32),
    )(parts, sb)

    return out.reshape(NP)[:N]
